# baseline (speedup 1.0000x reference)
import jax
import jax.numpy as jnp
from jax.experimental import pallas as pl

_K = 8
_MC = 512


def _chamfer_body(a1_ref, a2_ref, out_ref):
    b = pl.program_id(0)
    nbatch = pl.num_programs(0)
    f32 = jnp.float32
    a1 = a1_ref[0]
    a2t = a2_ref[0].T
    n = a1.shape[0]
    m = a2t.shape[1]
    n1 = jnp.sum(a1 * a1, axis=1, keepdims=True)
    n2 = jnp.sum(a2t * a2t, axis=0, keepdims=True)
    aug1 = jnp.concatenate(
        [-2.0 * a1, n1, jnp.ones((n, 1), f32), jnp.zeros((n, _K - 5), f32)],
        axis=1)
    aug2 = jnp.concatenate(
        [a2t, jnp.ones((1, m), f32), n2, jnp.zeros((_K - 5, m), f32)],
        axis=0)
    rowmin = None
    colmins = []
    for c in range(0, m, _MC):
        d = jnp.dot(aug1, aug2[:, c:c + _MC],
                    preferred_element_type=f32)
        rm = jnp.min(d, axis=1, keepdims=True)
        rowmin = rm if rowmin is None else jnp.minimum(rowmin, rm)
        colmins.append(jnp.min(d, axis=0, keepdims=True))
    rowmin = jnp.maximum(rowmin, 0.0)
    colmin = jnp.maximum(jnp.concatenate(colmins, axis=1), 0.0)
    s = (jnp.sum(rowmin, axis=(0, 1), keepdims=True) / (nbatch * n)
         + jnp.sum(colmin, axis=(0, 1), keepdims=True) / (nbatch * m))

    @pl.when(b == 0)
    def _():
        out_ref[...] = s

    @pl.when(b != 0)
    def _():
        out_ref[...] = out_ref[...] + s


def kernel(array1, array2):
    B, N, _ = array1.shape
    M = array2.shape[1]
    out = pl.pallas_call(
        _chamfer_body,
        grid=(B,),
        in_specs=[
            pl.BlockSpec((1, N, 3), lambda b: (b, 0, 0)),
            pl.BlockSpec((1, M, 3), lambda b: (b, 0, 0)),
        ],
        out_specs=pl.BlockSpec((1, 1), lambda b: (0, 0)),
        out_shape=jax.ShapeDtypeStruct((1, 1), jnp.float32),
    )(array1, array2)
    return out[0, 0]


# speedup vs baseline: 1.2953x; 1.2953x over previous
import jax
import jax.numpy as jnp
from jax.experimental import pallas as pl

_K = 8
_MC = 512


def _chamfer_body(a1_ref, a2_ref, out_ref):
    b = pl.program_id(0)
    nbatch = pl.num_programs(0)
    f32 = jnp.float32
    a1t = a1_ref[:, b, :]
    a2t = a2_ref[:, b, :]
    n = a1t.shape[1]
    m = a2t.shape[1]
    a1 = a1t.T
    n1 = jnp.sum(a1 * a1, axis=1, keepdims=True)
    n2 = jnp.sum(a2t * a2t, axis=0, keepdims=True)
    aug1 = jnp.concatenate(
        [-2.0 * a1, n1, jnp.ones((n, 1), f32), jnp.zeros((n, _K - 5), f32)],
        axis=1)
    aug2 = jnp.concatenate(
        [a2t, jnp.ones((1, m), f32), n2, jnp.zeros((_K - 5, m), f32)],
        axis=0)
    rowmin = None
    colmins = []
    for c in range(0, m, _MC):
        d = jnp.dot(aug1, aug2[:, c:c + _MC],
                    preferred_element_type=f32)
        rm = jnp.min(d, axis=1, keepdims=True)
        rowmin = rm if rowmin is None else jnp.minimum(rowmin, rm)
        colmins.append(jnp.min(d, axis=0, keepdims=True))
    rowmin = jnp.maximum(rowmin, 0.0)
    colmin = jnp.maximum(jnp.concatenate(colmins, axis=1), 0.0)
    s = (jnp.sum(rowmin, axis=(0, 1), keepdims=True) / (nbatch * n)
         + jnp.sum(colmin, axis=(0, 1), keepdims=True) / (nbatch * m))

    @pl.when(b == 0)
    def _():
        out_ref[...] = s

    @pl.when(b != 0)
    def _():
        out_ref[...] = out_ref[...] + s


def kernel(array1, array2):
    B, N, _ = array1.shape
    M = array2.shape[1]
    a1p = jnp.transpose(array1, (2, 0, 1))
    a2p = jnp.transpose(array2, (2, 0, 1))
    out = pl.pallas_call(
        _chamfer_body,
        grid=(B,),
        in_specs=[
            pl.BlockSpec((3, B, N), lambda b: (0, 0, 0)),
            pl.BlockSpec((3, B, M), lambda b: (0, 0, 0)),
        ],
        out_specs=pl.BlockSpec((1, 1), lambda b: (0, 0)),
        out_shape=jax.ShapeDtypeStruct((1, 1), jnp.float32),
    )(a1p, a2p)
    return out[0, 0]


# speedup vs baseline: 1.5083x; 1.1645x over previous
import jax
import jax.numpy as jnp
from jax import lax
from jax.experimental import pallas as pl

_K = 8
_MC = 512


def _chamfer_body(a1_ref, a2_ref, out_ref):
    b = pl.program_id(0)
    nbatch = pl.num_programs(0)
    f32 = jnp.float32
    a1t = a1_ref[:, b, :]
    a2t = a2_ref[:, b, :]
    n = a1t.shape[1]
    m = a2t.shape[1]
    n1 = jnp.sum(a1t * a1t, axis=0, keepdims=True)
    n2 = jnp.sum(a2t * a2t, axis=0, keepdims=True)
    aug1 = jnp.concatenate(
        [-2.0 * a1t, n1, jnp.ones((1, n), f32), jnp.zeros((_K - 5, n), f32)],
        axis=0)
    aug2 = jnp.concatenate(
        [a2t, jnp.ones((1, m), f32), n2, jnp.zeros((_K - 5, m), f32)],
        axis=0)
    rowmin = None
    colmins = []
    for c in range(0, m, _MC):
        d = lax.dot_general(aug1, aug2[:, c:c + _MC],
                            (((0,), (0,)), ((), ())),
                            preferred_element_type=f32)
        rm = jnp.min(d, axis=1, keepdims=True)
        rowmin = rm if rowmin is None else jnp.minimum(rowmin, rm)
        colmins.append(jnp.min(d, axis=0, keepdims=True))
    rowmin = jnp.maximum(rowmin, 0.0)
    colmin = jnp.maximum(jnp.concatenate(colmins, axis=1), 0.0)
    s = (jnp.sum(rowmin, axis=(0, 1), keepdims=True) / (nbatch * n)
         + jnp.sum(colmin, axis=(0, 1), keepdims=True) / (nbatch * m))

    @pl.when(b == 0)
    def _():
        out_ref[...] = s

    @pl.when(b != 0)
    def _():
        out_ref[...] = out_ref[...] + s


def kernel(array1, array2):
    B, N, _ = array1.shape
    M = array2.shape[1]
    a1p = jnp.transpose(array1, (2, 0, 1))
    a2p = jnp.transpose(array2, (2, 0, 1))
    out = pl.pallas_call(
        _chamfer_body,
        grid=(B,),
        in_specs=[
            pl.BlockSpec((3, B, N), lambda b: (0, 0, 0)),
            pl.BlockSpec((3, B, M), lambda b: (0, 0, 0)),
        ],
        out_specs=pl.BlockSpec((1, 1), lambda b: (0, 0)),
        out_shape=jax.ShapeDtypeStruct((1, 1), jnp.float32),
    )(a1p, a2p)
    return out[0, 0]


# speedup vs baseline: 1.6526x; 1.0957x over previous
import jax
import jax.numpy as jnp
from jax import lax
from jax.experimental import pallas as pl

_K = 8
_BU = 2


def _one_batch_dot(a1t, a2t):
    f32 = jnp.float32
    n = a1t.shape[1]
    m = a2t.shape[1]
    n1 = jnp.sum(a1t * a1t, axis=0, keepdims=True)
    n2 = jnp.sum(a2t * a2t, axis=0, keepdims=True)
    aug1 = jnp.concatenate(
        [-2.0 * a1t, n1, jnp.ones((1, n), f32), jnp.zeros((_K - 5, n), f32)],
        axis=0)
    aug2 = jnp.concatenate(
        [a2t, jnp.ones((1, m), f32), n2, jnp.zeros((_K - 5, m), f32)],
        axis=0)
    return lax.dot_general(aug1, aug2, (((0,), (0,)), ((), ())),
                           preferred_element_type=f32)


def _chamfer_body(a1_ref, a2_ref, out_ref):
    g = pl.program_id(0)
    nbatch = pl.num_programs(0) * _BU
    f32 = jnp.float32
    n = a1_ref.shape[2]
    m = a2_ref.shape[2]
    ds = [_one_batch_dot(a1_ref[:, _BU * g + u, :], a2_ref[:, _BU * g + u, :])
          for u in range(_BU)]
    s = jnp.zeros((1, 1), f32)
    for d in ds:
        rowmin = jnp.maximum(jnp.min(d, axis=1, keepdims=True), 0.0)
        colmin = jnp.maximum(jnp.min(d, axis=0, keepdims=True), 0.0)
        s = (s + jnp.sum(rowmin, axis=(0, 1), keepdims=True) / (nbatch * n)
             + jnp.sum(colmin, axis=(0, 1), keepdims=True) / (nbatch * m))

    @pl.when(g == 0)
    def _():
        out_ref[...] = s

    @pl.when(g != 0)
    def _():
        out_ref[...] = out_ref[...] + s


def kernel(array1, array2):
    B, N, _ = array1.shape
    M = array2.shape[1]
    a1p = jnp.transpose(array1, (2, 0, 1))
    a2p = jnp.transpose(array2, (2, 0, 1))
    out = pl.pallas_call(
        _chamfer_body,
        grid=(B // _BU,),
        in_specs=[
            pl.BlockSpec((3, B, N), lambda g: (0, 0, 0)),
            pl.BlockSpec((3, B, M), lambda g: (0, 0, 0)),
        ],
        out_specs=pl.BlockSpec((1, 1), lambda g: (0, 0)),
        out_shape=jax.ShapeDtypeStruct((1, 1), jnp.float32),
    )(a1p, a2p)
    return out[0, 0]


# speedup vs baseline: 1.7510x; 1.0595x over previous
import jax
import jax.numpy as jnp
from jax import lax
from jax.experimental import pallas as pl

_K = 8
_BU = 8


def _one_batch_dot(a1t, a2t):
    f32 = jnp.float32
    n = a1t.shape[1]
    m = a2t.shape[1]
    n1 = jnp.sum(a1t * a1t, axis=0, keepdims=True)
    n2 = jnp.sum(a2t * a2t, axis=0, keepdims=True)
    aug1 = jnp.concatenate(
        [-2.0 * a1t, n1, jnp.ones((1, n), f32), jnp.zeros((_K - 5, n), f32)],
        axis=0)
    aug2 = jnp.concatenate(
        [a2t, jnp.ones((1, m), f32), n2, jnp.zeros((_K - 5, m), f32)],
        axis=0)
    return lax.dot_general(aug1, aug2, (((0,), (0,)), ((), ())),
                           preferred_element_type=f32)


def _chamfer_body(a1_ref, a2_ref, out_ref):
    g = pl.program_id(0)
    nbatch = pl.num_programs(0) * _BU
    f32 = jnp.float32
    n = a1_ref.shape[2]
    m = a2_ref.shape[2]
    def reduce_d(d, s):
        rowmin = jnp.maximum(jnp.min(d, axis=1, keepdims=True), 0.0)
        colmin = jnp.maximum(jnp.min(d, axis=0, keepdims=True), 0.0)
        return (s + jnp.sum(rowmin, axis=(0, 1), keepdims=True) / (nbatch * n)
                + jnp.sum(colmin, axis=(0, 1), keepdims=True) / (nbatch * m))

    s = jnp.zeros((1, 1), f32)
    prev = None
    for u in range(_BU):
        d = _one_batch_dot(a1_ref[:, _BU * g + u, :], a2_ref[:, _BU * g + u, :])
        if prev is not None:
            s = reduce_d(prev, s)
        prev = d
    s = reduce_d(prev, s)

    @pl.when(g == 0)
    def _():
        out_ref[...] = s

    @pl.when(g != 0)
    def _():
        out_ref[...] = out_ref[...] + s


def kernel(array1, array2):
    B, N, _ = array1.shape
    M = array2.shape[1]
    a1p = jnp.transpose(array1, (2, 0, 1))
    a2p = jnp.transpose(array2, (2, 0, 1))
    out = pl.pallas_call(
        _chamfer_body,
        grid=(B // _BU,),
        in_specs=[
            pl.BlockSpec((3, B, N), lambda g: (0, 0, 0)),
            pl.BlockSpec((3, B, M), lambda g: (0, 0, 0)),
        ],
        out_specs=pl.BlockSpec((1, 1), lambda g: (0, 0)),
        out_shape=jax.ShapeDtypeStruct((1, 1), jnp.float32),
    )(a1p, a2p)
    return out[0, 0]
